# baseline (speedup 1.0000x reference)
import functools

import jax
import jax.numpy as jnp
from jax import lax
from jax.experimental import pallas as pl
from jax.experimental.pallas import tpu as pltpu
from jax.experimental.pallas import tpu_sc as plsc

_N = 10000
_E = 320000
_D = 128
_H1 = 64
_H2 = 128
_CLS = 10
_G = 64
_HH = _H1 // 2

_NC = 2
_NS = 16
_CHUNK = 128
_NBUF = 4
_EPT = 20480
_EPAD = _EPT * _NS
_ITERS = _EPT // _CHUNK
_NPAD = 10112
_RPS = _NPAD // _NS

_BN = 1000
_GRID = _N // _BN


def _sc_fused_body(y2, r2, b2c, src3, dst3, z32, z16, ones,
                   hout, aout, dout,
                   srcv, dstv, r0b, r1b, r2b, r3b, onesv, bbuf,
                   ysh, acc, dacc, s0, s1, s2, s3):
    cid = lax.axis_index("c")
    sid = lax.axis_index("s")
    rr = sid * _RPS
    pltpu.sync_copy(z32.at[pl.ds(rr, _RPS)], acc.at[pl.ds(rr, _RPS)])
    pltpu.sync_copy(z16.at[pl.ds(rr, _RPS)], dacc.at[pl.ds(rr, _RPS)])
    pltpu.sync_copy(ones, onesv)
    pltpu.sync_copy(b2c.at[cid], bbuf)
    pltpu.sync_copy(src3.at[sid], srcv)
    pltpu.sync_copy(dst3.at[sid], dstv)
    pltpu.sync_copy(y2.at[cid, pl.ds(rr, _RPS)], ysh.at[pl.ds(rr, _RPS)])
    plsc.subcore_barrier()
    rows = (r0b, r1b, r2b, r3b)
    sems = (s0, s1, s2, s3)

    def layer(i, carry):
        jj = i * _NBUF
        hs = [pltpu.async_copy(ysh.at[srcv.at[jj + b]], rows[b], sems[b])
              for b in range(_NBUF)]
        for b in range(_NBUF):
            hs[b].wait()
            pltpu.sync_copy(rows[b], acc.at[dstv.at[jj + b]], add=True)

        @pl.when(carry == 0)
        def _():
            for b in range(_NBUF):
                pltpu.sync_copy(onesv, dacc.at[dstv.at[jj + b]], add=True)

        return carry

    lax.fori_loop(0, _ITERS // _NBUF, layer, 0)
    plsc.subcore_barrier()

    b0 = bbuf[pl.ds(0, 16)]
    b1 = bbuf[pl.ds(16, 16)]
    for p, sz in enumerate((128, 128, 128, 128, _RPS - 512)):
        row0 = rr + p * 128
        pltpu.sync_copy(acc.at[pl.ds(row0, sz)], r0b.at[pl.ds(0, sz)])
        pltpu.sync_copy(dacc.at[pl.ds(row0, sz)], onesv.at[pl.ds(0, sz)])
        pltpu.sync_copy(r2.at[cid, pl.ds(row0, sz)], r1b.at[pl.ds(0, sz)])

        def hrow(r, carry):
            dgv = jnp.maximum(onesv[r, pl.ds(0, 16)], 1.0)
            h0 = jnp.maximum(r0b[r, pl.ds(0, 16)] / dgv + b0
                             + r1b[r, pl.ds(0, 16)], 0.0)
            h1 = jnp.maximum(r0b[r, pl.ds(16, 16)] / dgv + b1
                             + r1b[r, pl.ds(16, 16)], 0.0)
            r0b[r, pl.ds(0, 16)] = h0
            r0b[r, pl.ds(16, 16)] = h1
            return carry

        lax.fori_loop(0, sz, hrow, 0)
        pltpu.sync_copy(r0b.at[pl.ds(0, sz)], hout.at[cid, pl.ds(row0, sz)])
        pltpu.sync_copy(r0b.at[pl.ds(0, sz)], ysh.at[pl.ds(row0, sz)])
    pltpu.sync_copy(z32.at[pl.ds(rr, _RPS)], acc.at[pl.ds(rr, _RPS)])
    plsc.subcore_barrier()

    lax.fori_loop(0, _ITERS // _NBUF, layer, 1)
    plsc.subcore_barrier()
    pltpu.sync_copy(acc.at[pl.ds(rr, _RPS)], aout.at[cid, pl.ds(rr, _RPS)])

    @pl.when(cid == 0)
    def _():
        pltpu.sync_copy(dacc.at[pl.ds(rr, _RPS)], dout.at[pl.ds(rr, _RPS)])


@functools.lru_cache(maxsize=None)
def _get_sc_fused():
    return pl.kernel(
        _sc_fused_body,
        out_type=[
            jax.ShapeDtypeStruct((_NC, _NPAD, _HH), jnp.float32),
            jax.ShapeDtypeStruct((_NC, _NPAD, _HH), jnp.float32),
            jax.ShapeDtypeStruct((_NPAD, 16), jnp.float32),
        ],
        scratch_types=[
            pltpu.VMEM((_ITERS, _CHUNK), jnp.int32),
            pltpu.VMEM((_ITERS, _CHUNK), jnp.int32),
            pltpu.VMEM((_CHUNK, _HH), jnp.float32),
            pltpu.VMEM((_CHUNK, _HH), jnp.float32),
            pltpu.VMEM((_CHUNK, _HH), jnp.float32),
            pltpu.VMEM((_CHUNK, _HH), jnp.float32),
            pltpu.VMEM((_CHUNK, 16), jnp.float32),
            pltpu.VMEM((_HH,), jnp.float32),
            pltpu.VMEM_SHARED((_NPAD, _HH), jnp.float32),
            pltpu.VMEM_SHARED((_NPAD, _HH), jnp.float32),
            pltpu.VMEM_SHARED((_NPAD, 16), jnp.float32),
            pltpu.SemaphoreType.DMA,
            pltpu.SemaphoreType.DMA,
            pltpu.SemaphoreType.DMA,
            pltpu.SemaphoreType.DMA,
        ],
        mesh=plsc.VectorSubcoreMesh(core_axis_name="c", subcore_axis_name="s"),
        compiler_params=pltpu.CompilerParams(use_tc_tiling_on_sc=False),
    )


def _mm(a, b):
    return lax.dot_general(a, b, (((1,), (0,)), ((), ())),
                           preferred_element_type=jnp.float32)


def _pre_body(x_ref, wl_ref, wr_ref, y_ref, r_ref):
    xb = x_ref[...]
    yy = _mm(xb, wl_ref[...])
    y_ref[0] = yy[:, :_HH]
    y_ref[1] = yy[:, _HH:]
    rr = _mm(xb, wr_ref[...])
    r_ref[0] = rr[:, :_HH]
    r_ref[1] = rr[:, _HH:]


def _pre(x, w1lT, w1rT):
    return pl.pallas_call(
        _pre_body,
        grid=(_GRID,),
        in_specs=[
            pl.BlockSpec((_BN, _D), lambda i: (i, 0)),
            pl.BlockSpec((_D, _H1), lambda i: (0, 0)),
            pl.BlockSpec((_D, _H1), lambda i: (0, 0)),
        ],
        out_specs=[
            pl.BlockSpec((_NC, _BN, _HH), lambda i: (0, i, 0)),
            pl.BlockSpec((_NC, _BN, _HH), lambda i: (0, i, 0)),
        ],
        out_shape=[
            jax.ShapeDtypeStruct((_NC, _N, _HH), jnp.float32),
            jax.ShapeDtypeStruct((_NC, _N, _HH), jnp.float32),
        ],
    )(x, w1lT, w1rT)


def _post_body(a0_ref, a1_ref, d_ref, h0_ref, h1_ref, bat_ref,
               w2l_ref, w2r_ref, b2_ref, wfc_ref, bfc_ref, out_ref,
               pa_ref, ph_ref, cnt_ref):
    i = pl.program_id(0)

    @pl.when(i == 0)
    def _():
        pa_ref[...] = jnp.zeros_like(pa_ref)
        ph_ref[...] = jnp.zeros_like(ph_ref)
        cnt_ref[...] = jnp.zeros_like(cnt_ref)

    deg = jnp.maximum(d_ref[...][:, :1], 1.0)
    a2d = jnp.concatenate([a0_ref[0], a1_ref[0]], axis=1) / deg
    h = jnp.concatenate([h0_ref[0], h1_ref[0]], axis=1)
    p = (bat_ref[...] ==
         lax.broadcasted_iota(jnp.int32, (1, _G), 1)).astype(jnp.float32)
    ptd = (((0,), (0,)), ((), ()))
    pa_ref[...] += lax.dot_general(p, a2d, ptd,
                                   preferred_element_type=jnp.float32)
    ph_ref[...] += lax.dot_general(p, h, ptd,
                                   preferred_element_type=jnp.float32)
    cnt_ref[...] += lax.dot_general(p, jnp.ones((_BN, 128), jnp.float32), ptd,
                                    preferred_element_type=jnp.float32)

    @pl.when(i == _GRID - 1)
    def _():
        ccol = cnt_ref[...][:, :1]
        pooled = (_mm(pa_ref[...], w2l_ref[...]) +
                  _mm(ph_ref[...], w2r_ref[...]) +
                  ccol * b2_ref[...]) / jnp.maximum(ccol, 1.0)
        logits = _mm(pooled, wfc_ref[...]) + bfc_ref[...]
        m = jnp.max(logits, axis=1, keepdims=True)
        z = logits - m
        out_ref[...] = z - jnp.log(jnp.sum(jnp.exp(z), axis=1, keepdims=True))


def _post(a2, d, h2, bat, w2lT, w2rT, b2l, wfcT, bfc):
    return pl.pallas_call(
        _post_body,
        grid=(_GRID,),
        in_specs=[
            pl.BlockSpec((1, _BN, _HH), lambda i: (0, i, 0)),
            pl.BlockSpec((1, _BN, _HH), lambda i: (1, i, 0)),
            pl.BlockSpec((_BN, 16), lambda i: (i, 0)),
            pl.BlockSpec((1, _BN, _HH), lambda i: (0, i, 0)),
            pl.BlockSpec((1, _BN, _HH), lambda i: (1, i, 0)),
            pl.BlockSpec((_BN, 1), lambda i: (i, 0)),
            pl.BlockSpec((_H1, _H2), lambda i: (0, 0)),
            pl.BlockSpec((_H1, _H2), lambda i: (0, 0)),
            pl.BlockSpec((1, _H2), lambda i: (0, 0)),
            pl.BlockSpec((_H2, _CLS), lambda i: (0, 0)),
            pl.BlockSpec((1, _CLS), lambda i: (0, 0)),
        ],
        out_specs=pl.BlockSpec((_G, _CLS), lambda i: (0, 0)),
        out_shape=jax.ShapeDtypeStruct((_G, _CLS), jnp.float32),
        scratch_shapes=[
            pltpu.VMEM((_G, _H1), jnp.float32),
            pltpu.VMEM((_G, _H1), jnp.float32),
            pltpu.VMEM((_G, 128), jnp.float32),
        ],
    )(a2, a2, d, h2, h2, bat, w2lT, w2rT, b2l, wfcT, bfc)


def kernel(x, edge_index, batch, W1l, b1l, W1r, W2l, b2l, W2r, Wfc, bfc):
    src = edge_index[0]
    dst = edge_index[1]
    pad = _EPAD - _E
    src_t = jnp.concatenate([src, jnp.zeros((pad,), jnp.int32)]
                            ).reshape(_NS, _ITERS, _CHUNK)
    dst_t = jnp.concatenate([dst, jnp.full((pad,), _N, jnp.int32)]
                            ).reshape(_NS, _ITERS, _CHUNK)
    z32 = jnp.zeros((_NPAD, _HH), jnp.float32)
    z16 = jnp.zeros((_NPAD, 16), jnp.float32)
    ones = jnp.ones((_CHUNK, 16), jnp.float32)
    b2c = b1l.reshape(_NC, _HH)

    y2, r2 = _pre(x, W1l.T, W1r.T)
    zpad = jnp.zeros((_NC, _NPAD - _N, _HH), jnp.float32)
    y2 = jnp.concatenate([y2, zpad], axis=1)
    r2 = jnp.concatenate([r2, zpad], axis=1)
    h2, agg2, degp = _get_sc_fused()(y2, r2, b2c, src_t, dst_t, z32, z16, ones)
    return _post(agg2[:, :_N], degp[:_N], h2[:, :_N],
                 batch.reshape(_N, 1).astype(jnp.int32),
                 W2l.T, W2r.T, b2l.reshape(1, _H2), Wfc.T, bfc.reshape(1, _CLS))


# speedup vs baseline: 1.0287x; 1.0287x over previous
import functools

import jax
import jax.numpy as jnp
from jax import lax
from jax.experimental import pallas as pl
from jax.experimental.pallas import tpu as pltpu
from jax.experimental.pallas import tpu_sc as plsc

_N = 10000
_E = 320000
_D = 128
_H1 = 64
_H2 = 128
_CLS = 10
_G = 64
_HH = _H1 // 2

_NC = 2
_NS = 16
_CHUNK = 128
_NBUF = 8
_EPT = 20480
_EPAD = _EPT * _NS
_ITERS = _EPT // _CHUNK
_NPAD = 10112
_RPS = _NPAD // _NS

_BN = 1000
_GRID = _N // _BN


def _sc_fused_body(y2, r2, b2c, src3, dst3, z32, z16, ones,
                   hout, aout, dout,
                   srcv, dstv, r0b, r1b, r2b, r3b, r4b, r5b, r6b, r7b,
                   onesv, bbuf, ysh, acc, dacc,
                   s0, s1, s2, s3, s4, s5, s6, s7):
    cid = lax.axis_index("c")
    sid = lax.axis_index("s")
    rr = sid * _RPS
    pltpu.sync_copy(z32.at[pl.ds(rr, _RPS)], acc.at[pl.ds(rr, _RPS)])
    pltpu.sync_copy(z16.at[pl.ds(rr, _RPS)], dacc.at[pl.ds(rr, _RPS)])
    pltpu.sync_copy(ones, onesv)
    pltpu.sync_copy(b2c.at[cid], bbuf)
    pltpu.sync_copy(src3.at[sid], srcv)
    pltpu.sync_copy(dst3.at[sid], dstv)
    pltpu.sync_copy(y2.at[cid, pl.ds(rr, _RPS)], ysh.at[pl.ds(rr, _RPS)])
    plsc.subcore_barrier()
    rows = (r0b, r1b, r2b, r3b, r4b, r5b, r6b, r7b)
    sems = (s0, s1, s2, s3, s4, s5, s6, s7)

    def layer(i, carry):
        jj = i * _NBUF
        hs = [pltpu.async_copy(ysh.at[srcv.at[jj + b]], rows[b], sems[b])
              for b in range(_NBUF)]
        for b in range(_NBUF):
            hs[b].wait()
            pltpu.sync_copy(rows[b], acc.at[dstv.at[jj + b]], add=True)

        @pl.when(carry == 0)
        def _():
            for b in range(_NBUF):
                pltpu.sync_copy(onesv, dacc.at[dstv.at[jj + b]], add=True)

        return carry

    lax.fori_loop(0, _ITERS // _NBUF, layer, 0)
    plsc.subcore_barrier()

    b0 = bbuf[pl.ds(0, 16)]
    b1 = bbuf[pl.ds(16, 16)]
    for p, sz in enumerate((128, 128, 128, 128, _RPS - 512)):
        row0 = rr + p * 128
        pltpu.sync_copy(acc.at[pl.ds(row0, sz)], r0b.at[pl.ds(0, sz)])
        pltpu.sync_copy(dacc.at[pl.ds(row0, sz)], onesv.at[pl.ds(0, sz)])
        pltpu.sync_copy(r2.at[cid, pl.ds(row0, sz)], r1b.at[pl.ds(0, sz)])

        def hrow(r4, carry):
            for k in range(4):
                r = r4 * 4 + k
                dgv = jnp.maximum(onesv[r, pl.ds(0, 16)], 1.0)
                h0 = jnp.maximum(r0b[r, pl.ds(0, 16)] / dgv + b0
                                 + r1b[r, pl.ds(0, 16)], 0.0)
                h1 = jnp.maximum(r0b[r, pl.ds(16, 16)] / dgv + b1
                                 + r1b[r, pl.ds(16, 16)], 0.0)
                r0b[r, pl.ds(0, 16)] = h0
                r0b[r, pl.ds(16, 16)] = h1
            return carry

        lax.fori_loop(0, sz // 4, hrow, 0)
        pltpu.sync_copy(r0b.at[pl.ds(0, sz)], hout.at[cid, pl.ds(row0, sz)])
        pltpu.sync_copy(r0b.at[pl.ds(0, sz)], ysh.at[pl.ds(row0, sz)])
    pltpu.sync_copy(z32.at[pl.ds(rr, _RPS)], acc.at[pl.ds(rr, _RPS)])
    plsc.subcore_barrier()

    lax.fori_loop(0, _ITERS // _NBUF, layer, 1)
    plsc.subcore_barrier()
    pltpu.sync_copy(acc.at[pl.ds(rr, _RPS)], aout.at[cid, pl.ds(rr, _RPS)])

    @pl.when(cid == 0)
    def _():
        pltpu.sync_copy(dacc.at[pl.ds(rr, _RPS)], dout.at[pl.ds(rr, _RPS)])


@functools.lru_cache(maxsize=None)
def _get_sc_fused():
    return pl.kernel(
        _sc_fused_body,
        out_type=[
            jax.ShapeDtypeStruct((_NC, _NPAD, _HH), jnp.float32),
            jax.ShapeDtypeStruct((_NC, _NPAD, _HH), jnp.float32),
            jax.ShapeDtypeStruct((_NPAD, 16), jnp.float32),
        ],
        scratch_types=[
            pltpu.VMEM((_ITERS, _CHUNK), jnp.int32),
            pltpu.VMEM((_ITERS, _CHUNK), jnp.int32),
            pltpu.VMEM((_CHUNK, _HH), jnp.float32),
            pltpu.VMEM((_CHUNK, _HH), jnp.float32),
            pltpu.VMEM((_CHUNK, _HH), jnp.float32),
            pltpu.VMEM((_CHUNK, _HH), jnp.float32),
            pltpu.VMEM((_CHUNK, _HH), jnp.float32),
            pltpu.VMEM((_CHUNK, _HH), jnp.float32),
            pltpu.VMEM((_CHUNK, _HH), jnp.float32),
            pltpu.VMEM((_CHUNK, _HH), jnp.float32),
            pltpu.VMEM((_CHUNK, 16), jnp.float32),
            pltpu.VMEM((_HH,), jnp.float32),
            pltpu.VMEM_SHARED((_NPAD, _HH), jnp.float32),
            pltpu.VMEM_SHARED((_NPAD, _HH), jnp.float32),
            pltpu.VMEM_SHARED((_NPAD, 16), jnp.float32),
            pltpu.SemaphoreType.DMA,
            pltpu.SemaphoreType.DMA,
            pltpu.SemaphoreType.DMA,
            pltpu.SemaphoreType.DMA,
            pltpu.SemaphoreType.DMA,
            pltpu.SemaphoreType.DMA,
            pltpu.SemaphoreType.DMA,
            pltpu.SemaphoreType.DMA,
        ],
        mesh=plsc.VectorSubcoreMesh(core_axis_name="c", subcore_axis_name="s"),
        compiler_params=pltpu.CompilerParams(use_tc_tiling_on_sc=False),
    )


def _mm(a, b):
    return lax.dot_general(a, b, (((1,), (0,)), ((), ())),
                           preferred_element_type=jnp.float32)


def _pre_body(x_ref, wl_ref, wr_ref, y_ref, r_ref):
    xb = x_ref[...]
    yy = _mm(xb, wl_ref[...])
    y_ref[0] = yy[:, :_HH]
    y_ref[1] = yy[:, _HH:]
    rr = _mm(xb, wr_ref[...])
    r_ref[0] = rr[:, :_HH]
    r_ref[1] = rr[:, _HH:]


def _pre(x, w1lT, w1rT):
    return pl.pallas_call(
        _pre_body,
        grid=(_GRID,),
        in_specs=[
            pl.BlockSpec((_BN, _D), lambda i: (i, 0)),
            pl.BlockSpec((_D, _H1), lambda i: (0, 0)),
            pl.BlockSpec((_D, _H1), lambda i: (0, 0)),
        ],
        out_specs=[
            pl.BlockSpec((_NC, _BN, _HH), lambda i: (0, i, 0)),
            pl.BlockSpec((_NC, _BN, _HH), lambda i: (0, i, 0)),
        ],
        out_shape=[
            jax.ShapeDtypeStruct((_NC, _N, _HH), jnp.float32),
            jax.ShapeDtypeStruct((_NC, _N, _HH), jnp.float32),
        ],
    )(x, w1lT, w1rT)


def _post_body(a0_ref, a1_ref, d_ref, h0_ref, h1_ref, bat_ref,
               w2l_ref, w2r_ref, b2_ref, wfc_ref, bfc_ref, out_ref,
               pa_ref, ph_ref, cnt_ref):
    i = pl.program_id(0)

    @pl.when(i == 0)
    def _():
        pa_ref[...] = jnp.zeros_like(pa_ref)
        ph_ref[...] = jnp.zeros_like(ph_ref)
        cnt_ref[...] = jnp.zeros_like(cnt_ref)

    deg = jnp.maximum(d_ref[...][:, :1], 1.0)
    a2d = jnp.concatenate([a0_ref[0], a1_ref[0]], axis=1) / deg
    h = jnp.concatenate([h0_ref[0], h1_ref[0]], axis=1)
    p = (bat_ref[...] ==
         lax.broadcasted_iota(jnp.int32, (1, _G), 1)).astype(jnp.float32)
    ptd = (((0,), (0,)), ((), ()))
    pa_ref[...] += lax.dot_general(p, a2d, ptd,
                                   preferred_element_type=jnp.float32)
    ph_ref[...] += lax.dot_general(p, h, ptd,
                                   preferred_element_type=jnp.float32)
    cnt_ref[...] += lax.dot_general(p, jnp.ones((_BN, 128), jnp.float32), ptd,
                                    preferred_element_type=jnp.float32)

    @pl.when(i == _GRID - 1)
    def _():
        ccol = cnt_ref[...][:, :1]
        pooled = (_mm(pa_ref[...], w2l_ref[...]) +
                  _mm(ph_ref[...], w2r_ref[...]) +
                  ccol * b2_ref[...]) / jnp.maximum(ccol, 1.0)
        logits = _mm(pooled, wfc_ref[...]) + bfc_ref[...]
        m = jnp.max(logits, axis=1, keepdims=True)
        z = logits - m
        out_ref[...] = z - jnp.log(jnp.sum(jnp.exp(z), axis=1, keepdims=True))


def _post(a2, d, h2, bat, w2lT, w2rT, b2l, wfcT, bfc):
    return pl.pallas_call(
        _post_body,
        grid=(_GRID,),
        in_specs=[
            pl.BlockSpec((1, _BN, _HH), lambda i: (0, i, 0)),
            pl.BlockSpec((1, _BN, _HH), lambda i: (1, i, 0)),
            pl.BlockSpec((_BN, 16), lambda i: (i, 0)),
            pl.BlockSpec((1, _BN, _HH), lambda i: (0, i, 0)),
            pl.BlockSpec((1, _BN, _HH), lambda i: (1, i, 0)),
            pl.BlockSpec((_BN, 1), lambda i: (i, 0)),
            pl.BlockSpec((_H1, _H2), lambda i: (0, 0)),
            pl.BlockSpec((_H1, _H2), lambda i: (0, 0)),
            pl.BlockSpec((1, _H2), lambda i: (0, 0)),
            pl.BlockSpec((_H2, _CLS), lambda i: (0, 0)),
            pl.BlockSpec((1, _CLS), lambda i: (0, 0)),
        ],
        out_specs=pl.BlockSpec((_G, _CLS), lambda i: (0, 0)),
        out_shape=jax.ShapeDtypeStruct((_G, _CLS), jnp.float32),
        scratch_shapes=[
            pltpu.VMEM((_G, _H1), jnp.float32),
            pltpu.VMEM((_G, _H1), jnp.float32),
            pltpu.VMEM((_G, 128), jnp.float32),
        ],
    )(a2, a2, d, h2, h2, bat, w2lT, w2rT, b2l, wfcT, bfc)


def kernel(x, edge_index, batch, W1l, b1l, W1r, W2l, b2l, W2r, Wfc, bfc):
    src = edge_index[0]
    dst = edge_index[1]
    pad = _EPAD - _E
    src_t = jnp.concatenate([src, jnp.zeros((pad,), jnp.int32)]
                            ).reshape(_NS, _ITERS, _CHUNK)
    dst_t = jnp.concatenate([dst, jnp.full((pad,), _N, jnp.int32)]
                            ).reshape(_NS, _ITERS, _CHUNK)
    z32 = jnp.zeros((_NPAD, _HH), jnp.float32)
    z16 = jnp.zeros((_NPAD, 16), jnp.float32)
    ones = jnp.ones((_CHUNK, 16), jnp.float32)
    b2c = b1l.reshape(_NC, _HH)

    y2, r2 = _pre(x, W1l.T, W1r.T)
    zpad = jnp.zeros((_NC, _NPAD - _N, _HH), jnp.float32)
    y2 = jnp.concatenate([y2, zpad], axis=1)
    r2 = jnp.concatenate([r2, zpad], axis=1)
    h2, agg2, degp = _get_sc_fused()(y2, r2, b2c, src_t, dst_t, z32, z16, ones)
    return _post(agg2[:, :_N], degp[:_N], h2[:, :_N],
                 batch.reshape(_N, 1).astype(jnp.int32),
                 W2l.T, W2r.T, b2l.reshape(1, _H2), Wfc.T, bfc.reshape(1, _CLS))


# speedup vs baseline: 1.0727x; 1.0427x over previous
import functools

import jax
import jax.numpy as jnp
from jax import lax
from jax.experimental import pallas as pl
from jax.experimental.pallas import tpu as pltpu
from jax.experimental.pallas import tpu_sc as plsc

_N = 10000
_E = 320000
_D = 128
_H1 = 64
_H2 = 128
_CLS = 10
_G = 64
_HH = _H1 // 2

_NC = 2
_NS = 16
_NW = _NC * _NS
_CHUNK = 128
_NBUF = 8
_EPT = 20480
_EPAD = _EPT * _NS
_ITERS2 = _EPT // _CHUNK
_EPW = _EPAD // _NW
_ITERS = _EPW // _CHUNK
_NPAD = 10112
_RPS = _NPAD // _NS

_BN = 1000
_GRID = _N // _BN


def _sc_deg_body(dst3, z16, ones, deg, dstv, onesv, dacc, s0):
    cid = lax.axis_index("c")
    sid = lax.axis_index("s")
    wid = sid * _NC + cid
    rr = sid * _RPS
    pltpu.sync_copy(z16.at[pl.ds(rr, _RPS)], dacc.at[pl.ds(rr, _RPS)])
    pltpu.sync_copy(ones, onesv)
    pltpu.sync_copy(dst3.at[wid], dstv)
    plsc.subcore_barrier()

    def step(i, carry):
        pltpu.sync_copy(onesv, dacc.at[dstv.at[i]], add=True)
        return carry

    lax.fori_loop(0, _ITERS, step, 0)
    plsc.subcore_barrier()
    pltpu.sync_copy(dacc.at[pl.ds(rr, _RPS)], deg.at[cid, pl.ds(rr, _RPS)])


def _sc_agg_body(y2, src3, dst3, z32, out,
                 srcv, dstv, r0b, r1b, r2b, r3b, r4b, r5b, r6b, r7b,
                 ysh, acc, s0, s1, s2, s3, s4, s5, s6, s7):
    cid = lax.axis_index("c")
    sid = lax.axis_index("s")
    rr = sid * _RPS
    pltpu.sync_copy(z32.at[pl.ds(rr, _RPS)], acc.at[pl.ds(rr, _RPS)])
    pltpu.sync_copy(src3.at[sid], srcv)
    pltpu.sync_copy(dst3.at[sid], dstv)

    @pl.when(sid < _NS - 1)
    def _():
        pltpu.sync_copy(y2.at[cid, pl.ds(rr, _RPS)], ysh.at[pl.ds(rr, _RPS)])

    @pl.when(sid == _NS - 1)
    def _():
        last = (_NS - 1) * _RPS
        pltpu.sync_copy(y2.at[cid, pl.ds(last, _N - last)],
                        ysh.at[pl.ds(last, _N - last)])

    plsc.subcore_barrier()
    rows = (r0b, r1b, r2b, r3b, r4b, r5b, r6b, r7b)
    sems = (s0, s1, s2, s3, s4, s5, s6, s7)

    def step(i, carry):
        jj = i * _NBUF
        hs = [pltpu.async_copy(ysh.at[srcv.at[jj + b]], rows[b], sems[b])
              for b in range(_NBUF)]
        for b in range(_NBUF):
            hs[b].wait()
            pltpu.sync_copy(rows[b], acc.at[dstv.at[jj + b]], add=True)
        return carry

    lax.fori_loop(0, _ITERS2 // _NBUF, step, 0)
    plsc.subcore_barrier()
    pltpu.sync_copy(acc.at[pl.ds(rr, _RPS)], out.at[cid, pl.ds(rr, _RPS)])


@functools.lru_cache(maxsize=None)
def _get_sc_deg():
    return pl.kernel(
        _sc_deg_body,
        out_type=jax.ShapeDtypeStruct((_NC, _NPAD, 16), jnp.float32),
        scratch_types=[
            pltpu.VMEM((_ITERS, _CHUNK), jnp.int32),
            pltpu.VMEM((_CHUNK, 16), jnp.float32),
            pltpu.VMEM_SHARED((_NPAD, 16), jnp.float32),
            pltpu.SemaphoreType.DMA,
        ],
        mesh=plsc.VectorSubcoreMesh(core_axis_name="c", subcore_axis_name="s"),
        compiler_params=pltpu.CompilerParams(use_tc_tiling_on_sc=False),
    )


@functools.lru_cache(maxsize=None)
def _get_sc_agg():
    return pl.kernel(
        _sc_agg_body,
        out_type=jax.ShapeDtypeStruct((_NC, _NPAD, _HH), jnp.float32),
        scratch_types=[
            pltpu.VMEM((_ITERS2, _CHUNK), jnp.int32),
            pltpu.VMEM((_ITERS2, _CHUNK), jnp.int32),
            pltpu.VMEM((_CHUNK, _HH), jnp.float32),
            pltpu.VMEM((_CHUNK, _HH), jnp.float32),
            pltpu.VMEM((_CHUNK, _HH), jnp.float32),
            pltpu.VMEM((_CHUNK, _HH), jnp.float32),
            pltpu.VMEM((_CHUNK, _HH), jnp.float32),
            pltpu.VMEM((_CHUNK, _HH), jnp.float32),
            pltpu.VMEM((_CHUNK, _HH), jnp.float32),
            pltpu.VMEM((_CHUNK, _HH), jnp.float32),
            pltpu.VMEM_SHARED((_NPAD, _HH), jnp.float32),
            pltpu.VMEM_SHARED((_NPAD, _HH), jnp.float32),
            pltpu.SemaphoreType.DMA,
            pltpu.SemaphoreType.DMA,
            pltpu.SemaphoreType.DMA,
            pltpu.SemaphoreType.DMA,
            pltpu.SemaphoreType.DMA,
            pltpu.SemaphoreType.DMA,
            pltpu.SemaphoreType.DMA,
            pltpu.SemaphoreType.DMA,
        ],
        mesh=plsc.VectorSubcoreMesh(core_axis_name="c", subcore_axis_name="s"),
        compiler_params=pltpu.CompilerParams(use_tc_tiling_on_sc=False),
    )


def _mm(a, b):
    return lax.dot_general(a, b, (((1,), (0,)), ((), ())),
                           preferred_element_type=jnp.float32)


def _pre_body(x_ref, wl_ref, wr_ref, y_ref, r_ref):
    xb = x_ref[...]
    yy = _mm(xb, wl_ref[...])
    y_ref[0] = yy[:, :_HH]
    y_ref[1] = yy[:, _HH:]
    r_ref[...] = _mm(xb, wr_ref[...])


def _pre(x, w1lT, w1rT):
    return pl.pallas_call(
        _pre_body,
        grid=(_GRID,),
        in_specs=[
            pl.BlockSpec((_BN, _D), lambda i: (i, 0)),
            pl.BlockSpec((_D, _H1), lambda i: (0, 0)),
            pl.BlockSpec((_D, _H1), lambda i: (0, 0)),
        ],
        out_specs=[
            pl.BlockSpec((_NC, _BN, _HH), lambda i: (0, i, 0)),
            pl.BlockSpec((_BN, _H1), lambda i: (i, 0)),
        ],
        out_shape=[
            jax.ShapeDtypeStruct((_NC, _N, _HH), jnp.float32),
            jax.ShapeDtypeStruct((_N, _H1), jnp.float32),
        ],
    )(x, w1lT, w1rT)


def _mid_body(a0_ref, a1_ref, d0_ref, d1_ref, r_ref, b_ref, h_ref):
    deg = jnp.maximum(d0_ref[...][:, :1] + d1_ref[...][:, :1], 1.0)
    s = jnp.concatenate([a0_ref[0], a1_ref[0]], axis=1) / deg
    h = jnp.maximum(s + b_ref[...] + r_ref[...], 0.0)
    h_ref[0] = h[:, :_HH]
    h_ref[1] = h[:, _HH:]


def _mid(agg, d0, d1, r1, b1l):
    return pl.pallas_call(
        _mid_body,
        grid=(_GRID,),
        in_specs=[
            pl.BlockSpec((1, _BN, _HH), lambda i: (0, i, 0)),
            pl.BlockSpec((1, _BN, _HH), lambda i: (1, i, 0)),
            pl.BlockSpec((_BN, 16), lambda i: (i, 0)),
            pl.BlockSpec((_BN, 16), lambda i: (i, 0)),
            pl.BlockSpec((_BN, _H1), lambda i: (i, 0)),
            pl.BlockSpec((1, _H1), lambda i: (0, 0)),
        ],
        out_specs=pl.BlockSpec((_NC, _BN, _HH), lambda i: (0, i, 0)),
        out_shape=jax.ShapeDtypeStruct((_NC, _N, _HH), jnp.float32),
    )(agg, agg, d0, d1, r1, b1l)


def _post_body(a0_ref, a1_ref, d0_ref, d1_ref, h0_ref, h1_ref, bat_ref,
               w2l_ref, w2r_ref, b2_ref, wfc_ref, bfc_ref, out_ref,
               pa_ref, ph_ref, cnt_ref):
    i = pl.program_id(0)

    @pl.when(i == 0)
    def _():
        pa_ref[...] = jnp.zeros_like(pa_ref)
        ph_ref[...] = jnp.zeros_like(ph_ref)
        cnt_ref[...] = jnp.zeros_like(cnt_ref)

    deg = jnp.maximum(d0_ref[...][:, :1] + d1_ref[...][:, :1], 1.0)
    a2d = jnp.concatenate([a0_ref[0], a1_ref[0]], axis=1) / deg
    h = jnp.concatenate([h0_ref[0], h1_ref[0]], axis=1)
    p = (bat_ref[...] ==
         lax.broadcasted_iota(jnp.int32, (1, _G), 1)).astype(jnp.float32)
    ptd = (((0,), (0,)), ((), ()))
    pa_ref[...] += lax.dot_general(p, a2d, ptd,
                                   preferred_element_type=jnp.float32)
    ph_ref[...] += lax.dot_general(p, h, ptd,
                                   preferred_element_type=jnp.float32)
    cnt_ref[...] += lax.dot_general(p, jnp.ones((_BN, 128), jnp.float32), ptd,
                                    preferred_element_type=jnp.float32)

    @pl.when(i == _GRID - 1)
    def _():
        ccol = cnt_ref[...][:, :1]
        pooled = (_mm(pa_ref[...], w2l_ref[...]) +
                  _mm(ph_ref[...], w2r_ref[...]) +
                  ccol * b2_ref[...]) / jnp.maximum(ccol, 1.0)
        logits = _mm(pooled, wfc_ref[...]) + bfc_ref[...]
        m = jnp.max(logits, axis=1, keepdims=True)
        z = logits - m
        out_ref[...] = z - jnp.log(jnp.sum(jnp.exp(z), axis=1, keepdims=True))


def _post(a2, d0, d1, h2, bat, w2lT, w2rT, b2l, wfcT, bfc):
    return pl.pallas_call(
        _post_body,
        grid=(_GRID,),
        in_specs=[
            pl.BlockSpec((1, _BN, _HH), lambda i: (0, i, 0)),
            pl.BlockSpec((1, _BN, _HH), lambda i: (1, i, 0)),
            pl.BlockSpec((_BN, 16), lambda i: (i, 0)),
            pl.BlockSpec((_BN, 16), lambda i: (i, 0)),
            pl.BlockSpec((1, _BN, _HH), lambda i: (0, i, 0)),
            pl.BlockSpec((1, _BN, _HH), lambda i: (1, i, 0)),
            pl.BlockSpec((_BN, 1), lambda i: (i, 0)),
            pl.BlockSpec((_H1, _H2), lambda i: (0, 0)),
            pl.BlockSpec((_H1, _H2), lambda i: (0, 0)),
            pl.BlockSpec((1, _H2), lambda i: (0, 0)),
            pl.BlockSpec((_H2, _CLS), lambda i: (0, 0)),
            pl.BlockSpec((1, _CLS), lambda i: (0, 0)),
        ],
        out_specs=pl.BlockSpec((_G, _CLS), lambda i: (0, 0)),
        out_shape=jax.ShapeDtypeStruct((_G, _CLS), jnp.float32),
        scratch_shapes=[
            pltpu.VMEM((_G, _H1), jnp.float32),
            pltpu.VMEM((_G, _H1), jnp.float32),
            pltpu.VMEM((_G, 128), jnp.float32),
        ],
    )(a2, a2, d0, d1, h2, h2, bat, w2lT, w2rT, b2l, wfcT, bfc)


def kernel(x, edge_index, batch, W1l, b1l, W1r, W2l, b2l, W2r, Wfc, bfc):
    src = edge_index[0]
    dst = edge_index[1]
    pad = _EPAD - _E
    src_p = jnp.concatenate([src, jnp.zeros((pad,), jnp.int32)])
    dst_p = jnp.concatenate([dst, jnp.full((pad,), _N, jnp.int32)])
    src_t = src_p.reshape(_NS, _ITERS2, _CHUNK)
    dst_t = dst_p.reshape(_NS, _ITERS2, _CHUNK)
    dst_w = dst_p.reshape(_NW, _ITERS, _CHUNK)
    z32 = jnp.zeros((_NPAD, _HH), jnp.float32)
    z16 = jnp.zeros((_NPAD, 16), jnp.float32)
    ones = jnp.ones((_CHUNK, 16), jnp.float32)

    degp = _get_sc_deg()(dst_w, z16, ones)
    y2, r1 = _pre(x, W1l.T, W1r.T)
    agg1 = _get_sc_agg()(y2, src_t, dst_t, z32)
    d0 = degp[0, :_N]
    d1 = degp[1, :_N]
    h2 = _mid(agg1[:, :_N], d0, d1, r1, b1l.reshape(1, _H1))
    agg2 = _get_sc_agg()(h2, src_t, dst_t, z32)
    return _post(agg2[:, :_N], d0, d1, h2,
                 batch.reshape(_N, 1).astype(jnp.int32),
                 W2l.T, W2r.T, b2l.reshape(1, _H2), Wfc.T, bfc.reshape(1, _CLS))
